# baseline (speedup 1.0000x reference)
import functools

import jax
import jax.numpy as jnp
from jax import lax
from jax.experimental import pallas as pl
from jax.experimental.pallas import tpu as pltpu
from jax.experimental.pallas import tpu_sc as plsc

BATCH = 16384
DIM = 16
LANES = 16
MU = 7.0

_info = plsc.get_sparse_core_info()
NC = _info.num_cores
NS = _info.num_subcores
NW = NC * NS
B_PER_W = BATCH // NW
CHUNK = 128
N_CHUNKS = B_PER_W // CHUNK

_mesh = plsc.VectorSubcoreMesh(core_axis_name="c", subcore_axis_name="s")


@functools.partial(
    pl.kernel,
    mesh=_mesh,
    compiler_params=pltpu.CompilerParams(
        needs_layout_passes=False, use_tc_tiling_on_sc=False),
    out_type=jax.ShapeDtypeStruct((BATCH,), jnp.float32),
    scratch_types=[
        pltpu.VMEM((N_CHUNKS, CHUNK), jnp.int32),
        pltpu.VMEM((N_CHUNKS, CHUNK), jnp.int32),
        pltpu.VMEM((B_PER_W, DIM), jnp.float32),
        pltpu.VMEM((B_PER_W, DIM), jnp.float32),
        pltpu.VMEM((B_PER_W,), jnp.float32),
        pltpu.VMEM((B_PER_W,), jnp.float32),
        pltpu.VMEM((B_PER_W,), jnp.float32),
        pltpu.SemaphoreType.DMA,
    ],
)
def _mf_sc(uidx_hbm, iidx_hbm, ue_hbm, ie_hbm, ub_hbm, ib_hbm, out_hbm,
           uix, iix, urows, irows, ubv, ibv, outv, sem):
    wid = lax.axis_index("s") * NC + lax.axis_index("c")
    base = wid * B_PER_W

    cps = []
    for c in range(N_CHUNKS):
        off = base + c * CHUNK
        cps.append(pltpu.async_copy(uidx_hbm.at[pl.ds(off, CHUNK)], uix.at[c], sem))
        cps.append(pltpu.async_copy(iidx_hbm.at[pl.ds(off, CHUNK)], iix.at[c], sem))
    for cp in cps:
        cp.wait()

    cps = []
    for c in range(N_CHUNKS):
        rows = pl.ds(c * CHUNK, CHUNK)
        cps.append(pltpu.async_copy(ue_hbm.at[uix.at[c]], urows.at[rows, :], sem))
        cps.append(pltpu.async_copy(ie_hbm.at[iix.at[c]], irows.at[rows, :], sem))
        cps.append(pltpu.async_copy(ub_hbm.at[uix.at[c]], ubv.at[rows], sem))
        cps.append(pltpu.async_copy(ib_hbm.at[iix.at[c]], ibv.at[rows], sem))
    for cp in cps:
        cp.wait()

    lane = lax.iota(jnp.int32, LANES)

    def group(g, carry):
        rr = g * LANES + lane
        acc = jnp.zeros((LANES,), jnp.float32)
        for d in range(DIM):
            dd = jnp.full((LANES,), d, jnp.int32)
            u = plsc.load_gather(urows, [rr, dd])
            v = plsc.load_gather(irows, [rr, dd])
            acc = acc + u * v
        o = pl.multiple_of(g * LANES, LANES)
        outv[pl.ds(o, LANES)] = (acc + ubv[pl.ds(o, LANES)]
                                 + ibv[pl.ds(o, LANES)] + MU)
        return carry

    lax.fori_loop(0, B_PER_W // LANES, group, 0)

    pltpu.sync_copy(outv, out_hbm.at[pl.ds(base, B_PER_W)])


def kernel(user_indices, item_indices, user_embedding, item_embedding,
           user_bias, item_bias):
    ui = user_indices.astype(jnp.int32)
    ii = item_indices.astype(jnp.int32)
    ub = user_bias.reshape(-1)
    ib = item_bias.reshape(-1)
    return _mf_sc(ui, ii, user_embedding, item_embedding, ub, ib)
